# baseline (device time: 125511 ns/iter reference)
import jax
import jax.numpy as jnp
from jax import lax
from jax.experimental import pallas as pl
from jax.experimental.pallas import tpu as pltpu

N_DEV = 32
HEADS_PER = 8
DH = 128
SQ = 1024
D_MODEL = 1024
BLK = 64
SCALE = 0.08838834764831843

N_STEPS = 5
HALF = [512, 256, 128, 64, 32]
OFFS = [0, 512, 768, 896, 960]
COMM_ROWS = 992


def _lid(cx, cy, cz):
    return 8 * cz + 2 * cy + jnp.bitwise_xor(cx, jnp.bitwise_and(cy, 1))


def _allreduce_body(p_ref, out_ref, comm_ref, agcomm_ref, stage_ref, rs_send,
                    rs_recv, ag_send, ag_recv):
    my = lax.axis_index("i")
    z = my // 8
    p = my % 8
    y = p // 2
    x = (p + y) % 2

    steps = [
        (_lid(1 - x, y, z), x),
        (_lid(x, jnp.bitwise_xor(y, 1), z), jnp.bitwise_and(y, 1)),
        (_lid(x, y, jnp.bitwise_xor(z, 1)), jnp.bitwise_and(z, 1)),
        (_lid(x, jnp.bitwise_xor(y, 2), z), y // 2),
        (_lid(x, y, jnp.bitwise_xor(z, 2)), z // 2),
    ]

    barrier_sem = pltpu.get_barrier_semaphore()
    for partner, _ in steps:
        pl.semaphore_signal(
            barrier_sem, inc=1,
            device_id=(partner,), device_id_type=pl.DeviceIdType.MESH,
        )
    pl.semaphore_wait(barrier_sem, N_STEPS)

    out_ref[...] = p_ref[...]

    seg_start = 0
    for k in range(N_STEPS):
        partner, b = steps[k]
        half = HALF[k]
        send_start = seg_start + (1 - b) * half
        keep_start = seg_start + b * half
        stage_ref[pl.ds(0, half), :] = out_ref[
            pl.ds(send_start, half), :].astype(jnp.bfloat16)
        rdma = pltpu.make_async_remote_copy(
            src_ref=stage_ref.at[pl.ds(0, half)],
            dst_ref=comm_ref.at[pl.ds(OFFS[k], half)],
            send_sem=rs_send.at[k],
            recv_sem=rs_recv.at[k],
            device_id=(partner,),
            device_id_type=pl.DeviceIdType.MESH,
        )
        rdma.start()
        rdma.wait()
        acc = out_ref[pl.ds(keep_start, half), :] + comm_ref[
            pl.ds(OFFS[k], half), :].astype(jnp.float32)
        out_ref[pl.ds(keep_start, half), :] = acc
        seg_start = keep_start

    for k in reversed(range(N_STEPS)):
        partner, b = steps[k]
        size = HALF[k]
        stage_ref[pl.ds(0, size), :] = out_ref[
            pl.ds(seg_start, size), :].astype(jnp.bfloat16)
        rdma = pltpu.make_async_remote_copy(
            src_ref=stage_ref.at[pl.ds(0, size)],
            dst_ref=agcomm_ref.at[pl.ds(OFFS[k], size)],
            send_sem=ag_send.at[k],
            recv_sem=ag_recv.at[k],
            device_id=(partner,),
            device_id_type=pl.DeviceIdType.MESH,
        )
        rdma.start()
        rdma.wait()
        partner_start = seg_start + (1 - 2 * b) * size
        out_ref[pl.ds(partner_start, size), :] = agcomm_ref[
            pl.ds(OFFS[k], size), :].astype(jnp.float32)
        seg_start = seg_start - b * size


def _butterfly_allreduce(partial):
    return pl.pallas_call(
        _allreduce_body,
        out_shape=jax.ShapeDtypeStruct((SQ, D_MODEL), jnp.float32),
        in_specs=[pl.BlockSpec(memory_space=pltpu.VMEM)],
        out_specs=pl.BlockSpec(memory_space=pltpu.VMEM),
        scratch_shapes=[
            pltpu.VMEM((COMM_ROWS, D_MODEL), jnp.bfloat16),
            pltpu.VMEM((COMM_ROWS, D_MODEL), jnp.bfloat16),
            pltpu.VMEM((HALF[0], D_MODEL), jnp.bfloat16),
            pltpu.SemaphoreType.DMA((N_STEPS,)),
            pltpu.SemaphoreType.DMA((N_STEPS,)),
            pltpu.SemaphoreType.DMA((N_STEPS,)),
            pltpu.SemaphoreType.DMA((N_STEPS,)),
        ],
        compiler_params=pltpu.CompilerParams(collective_id=0),
    )(partial)


def _compute_body(x_ref, wq_ref, k_ref, v_ref, wo_ref, out_ref):
    h = pl.program_id(0)
    qh = jnp.dot(
        x_ref[...], wq_ref[...], preferred_element_type=jnp.float32
    ).astype(jnp.bfloat16)
    s = lax.dot_general(
        qh, k_ref[0], (((1,), (1,)), ((), ())),
        preferred_element_type=jnp.float32,
    ) * SCALE
    rb = lax.broadcasted_iota(jnp.int32, (SQ, SQ), 0) // BLK
    cb = lax.broadcasted_iota(jnp.int32, (SQ, SQ), 1) // BLK
    s = jnp.where(cb <= rb, s, -1e9)
    m = jnp.max(s, axis=1, keepdims=True)
    e = jnp.exp(s - m)
    w = (e / jnp.sum(e, axis=1, keepdims=True)).astype(jnp.bfloat16)
    ctx = jnp.dot(
        w, v_ref[0], preferred_element_type=jnp.float32
    ).astype(jnp.bfloat16)
    contrib = jnp.dot(
        ctx, wo_ref[0], preferred_element_type=jnp.float32
    )

    @pl.when(h == 0)
    def _():
        out_ref[...] = contrib

    @pl.when(h != 0)
    def _():
        out_ref[...] += contrib


def _compute_partial(xb, Wqb, Kb, Vb, Wob):
    return pl.pallas_call(
        _compute_body,
        grid=(HEADS_PER,),
        out_shape=jax.ShapeDtypeStruct((SQ, D_MODEL), jnp.float32),
        in_specs=[
            pl.BlockSpec((SQ, D_MODEL), lambda h: (0, 0)),
            pl.BlockSpec((D_MODEL, DH), lambda h: (0, h)),
            pl.BlockSpec((1, SQ, DH), lambda h: (h, 0, 0)),
            pl.BlockSpec((1, SQ, DH), lambda h: (h, 0, 0)),
            pl.BlockSpec((1, DH, D_MODEL), lambda h: (h, 0, 0)),
        ],
        out_specs=pl.BlockSpec((SQ, D_MODEL), lambda h: (0, 0)),
    )(xb, Wqb, Kb, Vb, Wob)


def kernel(x, Wq, K_ext, V_ext, Wo):
    my = lax.axis_index("i")

    xb = x[0].astype(jnp.bfloat16)
    Wqb = Wq.astype(jnp.bfloat16)

    K = lax.dynamic_slice_in_dim(K_ext[0], my * HEADS_PER, HEADS_PER, axis=1)
    V = lax.dynamic_slice_in_dim(V_ext[0], my * HEADS_PER, HEADS_PER, axis=1)
    Kb = K.transpose(1, 0, 2).astype(jnp.bfloat16)
    Vb = V.transpose(1, 0, 2).astype(jnp.bfloat16)
    Wob = Wo.reshape(HEADS_PER, DH, D_MODEL).astype(jnp.bfloat16)

    partial = _compute_partial(xb, Wqb, Kb, Vb, Wob)
    out = _butterfly_allreduce(partial)
    return out[None, :, :]


# device time: 109022 ns/iter; 1.1512x vs baseline; 1.1512x over previous
import jax
import jax.numpy as jnp
from jax import lax
from jax.experimental import pallas as pl
from jax.experimental.pallas import tpu as pltpu

N_DEV = 32
HEADS_PER = 8
DH = 128
SQ = 1024
D_MODEL = 1024
BLK = 64
SCALE = 0.08838834764831843

N_STEPS = 5
ROWS_HALF = SQ // 2
HALF = [256, 128, 64, 32, 16]
OFFS = [0, 256, 384, 448, 480]
COMM_ROWS = 496
ORDER_A = [0, 1, 2, 3, 4]
ORDER_B = [1, 0, 4, 2, 3]


def _lid(cx, cy, cz):
    return 8 * cz + 2 * cy + jnp.bitwise_xor(cx, jnp.bitwise_and(cy, 1))


def _allreduce_body(p_ref, out_ref,
                    commA, agcommA, stageA, commB, agcommB, stageB,
                    rsA_send, rsA_recv, agA_send, agA_recv,
                    rsB_send, rsB_recv, agB_send, agB_recv):
    my = lax.axis_index("i")
    z = my // 8
    p = my % 8
    y = p // 2
    x = (p + y) % 2

    dims = [
        (_lid(1 - x, y, z), x),
        (_lid(x, jnp.bitwise_xor(y, 1), z), jnp.bitwise_and(y, 1)),
        (_lid(x, y, jnp.bitwise_xor(z, 1)), jnp.bitwise_and(z, 1)),
        (_lid(x, jnp.bitwise_xor(y, 2), z), y // 2),
        (_lid(x, y, jnp.bitwise_xor(z, 2)), z // 2),
    ]

    barrier_sem = pltpu.get_barrier_semaphore()
    for partner, _ in dims:
        pl.semaphore_signal(
            barrier_sem, inc=1,
            device_id=(partner,), device_id_type=pl.DeviceIdType.MESH,
        )
    pl.semaphore_wait(barrier_sem, N_STEPS)

    out_ref[...] = p_ref[...]

    def rs_rdma(base, seg_start, k, order, comm, stage, send_sems, recv_sems):
        partner, b = dims[order[k]]
        half = HALF[k]
        send_start = pl.multiple_of(base + seg_start + (1 - b) * half, 16)
        keep_start = seg_start + b * half
        stage[pl.ds(0, half), :] = out_ref[
            pl.ds(send_start, half), :].astype(jnp.bfloat16)
        rdma = pltpu.make_async_remote_copy(
            src_ref=stage.at[pl.ds(0, half)],
            dst_ref=comm.at[pl.ds(OFFS[k], half)],
            send_sem=send_sems.at[k],
            recv_sem=recv_sems.at[k],
            device_id=(partner,),
            device_id_type=pl.DeviceIdType.MESH,
        )
        rdma.start()
        return rdma, keep_start

    def rs_accum(base, keep_start, k, comm):
        half = HALF[k]
        start = pl.multiple_of(base + keep_start, 16)
        acc = out_ref[pl.ds(start, half), :] + comm[
            pl.ds(OFFS[k], half), :].astype(jnp.float32)
        out_ref[pl.ds(start, half), :] = acc

    segA = 0
    segB = 0
    for k in range(N_STEPS):
        rdmaA, keepA = rs_rdma(0, segA, k, ORDER_A, commA, stageA,
                               rsA_send, rsA_recv)
        rdmaB, keepB = rs_rdma(ROWS_HALF, segB, k, ORDER_B, commB, stageB,
                               rsB_send, rsB_recv)
        rdmaA.wait()
        rs_accum(0, keepA, k, commA)
        rdmaB.wait()
        rs_accum(ROWS_HALF, keepB, k, commB)
        segA = keepA
        segB = keepB

    def ag_rdma(base, seg_start, k, order, agcomm, stage, send_sems,
                recv_sems):
        partner, b = dims[order[k]]
        size = HALF[k]
        src_start = pl.multiple_of(base + seg_start, 16)
        stage[pl.ds(0, size), :] = out_ref[
            pl.ds(src_start, size), :].astype(jnp.bfloat16)
        rdma = pltpu.make_async_remote_copy(
            src_ref=stage.at[pl.ds(0, size)],
            dst_ref=agcomm.at[pl.ds(OFFS[k], size)],
            send_sem=send_sems.at[k],
            recv_sem=recv_sems.at[k],
            device_id=(partner,),
            device_id_type=pl.DeviceIdType.MESH,
        )
        rdma.start()
        return rdma

    def ag_store(base, seg_start, k, order, agcomm):
        _, b = dims[order[k]]
        size = HALF[k]
        partner_start = pl.multiple_of(
            base + seg_start + (1 - 2 * b) * size, 16)
        out_ref[pl.ds(partner_start, size), :] = agcomm[
            pl.ds(OFFS[k], size), :].astype(jnp.float32)
        return seg_start - b * size

    for k in reversed(range(N_STEPS)):
        rdmaA = ag_rdma(0, segA, k, ORDER_A, agcommA, stageA,
                        agA_send, agA_recv)
        rdmaB = ag_rdma(ROWS_HALF, segB, k, ORDER_B, agcommB, stageB,
                        agB_send, agB_recv)
        rdmaA.wait()
        segA = ag_store(0, segA, k, ORDER_A, agcommA)
        rdmaB.wait()
        segB = ag_store(ROWS_HALF, segB, k, ORDER_B, agcommB)


def _butterfly_allreduce(partial):
    bf16_buf = lambda rows: pltpu.VMEM((rows, D_MODEL), jnp.bfloat16)
    sems = pltpu.SemaphoreType.DMA((N_STEPS,))
    return pl.pallas_call(
        _allreduce_body,
        out_shape=jax.ShapeDtypeStruct((SQ, D_MODEL), jnp.float32),
        in_specs=[pl.BlockSpec(memory_space=pltpu.VMEM)],
        out_specs=pl.BlockSpec(memory_space=pltpu.VMEM),
        scratch_shapes=[
            bf16_buf(COMM_ROWS), bf16_buf(COMM_ROWS), bf16_buf(HALF[0]),
            bf16_buf(COMM_ROWS), bf16_buf(COMM_ROWS), bf16_buf(HALF[0]),
            sems, sems, sems, sems,
            sems, sems, sems, sems,
        ],
        compiler_params=pltpu.CompilerParams(collective_id=0),
    )(partial)


def _compute_body(x_ref, wq_ref, k_ref, v_ref, wo_ref, out_ref, bias_ref):
    h = pl.program_id(0)

    @pl.when(h == 0)
    def _():
        rb = lax.broadcasted_iota(jnp.int32, (SQ, SQ), 0) // BLK
        cb = lax.broadcasted_iota(jnp.int32, (SQ, SQ), 1) // BLK
        bias_ref[...] = jnp.where(cb <= rb, 0.0, -1e9).astype(jnp.float32)

    qh = jnp.dot(
        x_ref[...], wq_ref[...], preferred_element_type=jnp.float32
    ).astype(jnp.bfloat16)
    s = lax.dot_general(
        qh, k_ref[0], (((1,), (1,)), ((), ())),
        preferred_element_type=jnp.float32,
    ) + bias_ref[...]
    m = jnp.max(s, axis=1, keepdims=True)
    e = jnp.exp(s - m)
    w = (e * (1.0 / jnp.sum(e, axis=1, keepdims=True))).astype(jnp.bfloat16)
    ctx = jnp.dot(
        w, v_ref[0], preferred_element_type=jnp.float32
    ).astype(jnp.bfloat16)
    contrib = jnp.dot(
        ctx, wo_ref[0], preferred_element_type=jnp.float32
    )

    @pl.when(h == 0)
    def _():
        out_ref[...] = contrib

    @pl.when(h != 0)
    def _():
        out_ref[...] += contrib


def _compute_partial(xb, Wqb, Kb, Vb, Wob):
    return pl.pallas_call(
        _compute_body,
        grid=(HEADS_PER,),
        out_shape=jax.ShapeDtypeStruct((SQ, D_MODEL), jnp.float32),
        in_specs=[
            pl.BlockSpec((SQ, D_MODEL), lambda h: (0, 0)),
            pl.BlockSpec((D_MODEL, DH), lambda h: (0, h)),
            pl.BlockSpec((1, SQ, DH), lambda h: (h, 0, 0)),
            pl.BlockSpec((1, SQ, DH), lambda h: (h, 0, 0)),
            pl.BlockSpec((1, DH, D_MODEL), lambda h: (h, 0, 0)),
        ],
        out_specs=pl.BlockSpec((SQ, D_MODEL), lambda h: (0, 0)),
        scratch_shapes=[pltpu.VMEM((SQ, SQ), jnp.float32)],
    )(xb, Wqb, Kb, Vb, Wob)


def kernel(x, Wq, K_ext, V_ext, Wo):
    my = lax.axis_index("i")

    xb = x[0].astype(jnp.bfloat16)
    Wqb = (Wq * SCALE).astype(jnp.bfloat16)

    K = lax.dynamic_slice_in_dim(K_ext[0], my * HEADS_PER, HEADS_PER, axis=1)
    V = lax.dynamic_slice_in_dim(V_ext[0], my * HEADS_PER, HEADS_PER, axis=1)
    Kb = K.transpose(1, 0, 2).astype(jnp.bfloat16)
    Vb = V.transpose(1, 0, 2).astype(jnp.bfloat16)
    Wob = Wo.reshape(HEADS_PER, DH, D_MODEL).astype(jnp.bfloat16)

    partial = _compute_partial(xb, Wqb, Kb, Vb, Wob)
    out = _butterfly_allreduce(partial)
    return out[None, :, :]


# device time: 104517 ns/iter; 1.2009x vs baseline; 1.0431x over previous
import jax
import jax.numpy as jnp
from jax import lax
from jax.experimental import pallas as pl
from jax.experimental.pallas import tpu as pltpu

N_DEV = 32
HEADS_PER = 8
DH = 128
SQ = 1024
D_MODEL = 1024
BLK = 64
SCALE = 0.08838834764831843

N_STEPS = 5
ROWS_HALF = SQ // 2
HALF = [256, 128, 64, 32, 16]
OFFS = [0, 256, 384, 448, 480]
COMM_ROWS = 496
ORDER_A = [0, 1, 2, 3, 4]
ORDER_B = [1, 0, 4, 2, 3]


def _lid(cx, cy, cz):
    return 8 * cz + 2 * cy + jnp.bitwise_xor(cx, jnp.bitwise_and(cy, 1))


def _allreduce_body(p_ref, out_ref,
                    commA, agcommA, stageA, commB, agcommB, stageB,
                    rsA_send, rsA_recv, agA_send, agA_recv,
                    rsB_send, rsB_recv, agB_send, agB_recv):
    my = lax.axis_index("i")
    z = my // 8
    p = my % 8
    y = p // 2
    x = (p + y) % 2

    dims = [
        (_lid(1 - x, y, z), x),
        (_lid(x, jnp.bitwise_xor(y, 1), z), jnp.bitwise_and(y, 1)),
        (_lid(x, y, jnp.bitwise_xor(z, 1)), jnp.bitwise_and(z, 1)),
        (_lid(x, jnp.bitwise_xor(y, 2), z), y // 2),
        (_lid(x, y, jnp.bitwise_xor(z, 2)), z // 2),
    ]

    barrier_sem = pltpu.get_barrier_semaphore()
    for partner, _ in dims:
        pl.semaphore_signal(
            barrier_sem, inc=1,
            device_id=(partner,), device_id_type=pl.DeviceIdType.MESH,
        )
    pl.semaphore_wait(barrier_sem, N_STEPS)

    out_ref[...] = p_ref[...]

    def rs_rdma(base, seg_start, k, order, comm, stage, send_sems, recv_sems):
        partner, b = dims[order[k]]
        half = HALF[k]
        send_start = pl.multiple_of(base + seg_start + (1 - b) * half, 16)
        keep_start = seg_start + b * half
        stage[pl.ds(0, half), :] = out_ref[
            pl.ds(send_start, half), :].astype(jnp.bfloat16)
        rdma = pltpu.make_async_remote_copy(
            src_ref=stage.at[pl.ds(0, half)],
            dst_ref=comm.at[pl.ds(OFFS[k], half)],
            send_sem=send_sems.at[k],
            recv_sem=recv_sems.at[k],
            device_id=(partner,),
            device_id_type=pl.DeviceIdType.MESH,
        )
        rdma.start()
        return rdma, keep_start

    def rs_accum(base, keep_start, k, comm):
        half = HALF[k]
        start = pl.multiple_of(base + keep_start, 16)
        acc = out_ref[pl.ds(start, half), :] + comm[
            pl.ds(OFFS[k], half), :].astype(jnp.float32)
        out_ref[pl.ds(start, half), :] = acc

    segA = 0
    segB = 0
    for k in range(N_STEPS):
        rdmaA, keepA = rs_rdma(0, segA, k, ORDER_A, commA, stageA,
                               rsA_send, rsA_recv)
        rdmaB, keepB = rs_rdma(ROWS_HALF, segB, k, ORDER_B, commB, stageB,
                               rsB_send, rsB_recv)
        rdmaA.wait()
        rs_accum(0, keepA, k, commA)
        rdmaB.wait()
        rs_accum(ROWS_HALF, keepB, k, commB)
        segA = keepA
        segB = keepB

    def ag_rdma(base, seg_start, k, order, agcomm, stage, send_sems,
                recv_sems):
        partner, b = dims[order[k]]
        size = HALF[k]
        src_start = pl.multiple_of(base + seg_start, 16)
        stage[pl.ds(0, size), :] = out_ref[
            pl.ds(src_start, size), :].astype(jnp.bfloat16)
        rdma = pltpu.make_async_remote_copy(
            src_ref=stage.at[pl.ds(0, size)],
            dst_ref=agcomm.at[pl.ds(OFFS[k], size)],
            send_sem=send_sems.at[k],
            recv_sem=recv_sems.at[k],
            device_id=(partner,),
            device_id_type=pl.DeviceIdType.MESH,
        )
        rdma.start()
        return rdma

    def ag_store(base, seg_start, k, order, agcomm):
        _, b = dims[order[k]]
        size = HALF[k]
        partner_start = pl.multiple_of(
            base + seg_start + (1 - 2 * b) * size, 16)
        out_ref[pl.ds(partner_start, size), :] = agcomm[
            pl.ds(OFFS[k], size), :].astype(jnp.float32)
        return seg_start - b * size

    for k in reversed(range(N_STEPS)):
        rdmaA = ag_rdma(0, segA, k, ORDER_A, agcommA, stageA,
                        agA_send, agA_recv)
        rdmaB = ag_rdma(ROWS_HALF, segB, k, ORDER_B, agcommB, stageB,
                        agB_send, agB_recv)
        rdmaA.wait()
        segA = ag_store(0, segA, k, ORDER_A, agcommA)
        rdmaB.wait()
        segB = ag_store(ROWS_HALF, segB, k, ORDER_B, agcommB)


def _butterfly_allreduce(partial):
    bf16_buf = lambda rows: pltpu.VMEM((rows, D_MODEL), jnp.bfloat16)
    sems = pltpu.SemaphoreType.DMA((N_STEPS,))
    return pl.pallas_call(
        _allreduce_body,
        out_shape=jax.ShapeDtypeStruct((SQ, D_MODEL), jnp.float32),
        in_specs=[pl.BlockSpec(memory_space=pltpu.VMEM)],
        out_specs=pl.BlockSpec(memory_space=pltpu.VMEM),
        scratch_shapes=[
            bf16_buf(COMM_ROWS), bf16_buf(COMM_ROWS), bf16_buf(HALF[0]),
            bf16_buf(COMM_ROWS), bf16_buf(COMM_ROWS), bf16_buf(HALF[0]),
            sems, sems, sems, sems,
            sems, sems, sems, sems,
        ],
        compiler_params=pltpu.CompilerParams(collective_id=0),
    )(partial)


def _softmax_rows(s):
    m = jnp.max(s, axis=1, keepdims=True)
    e = jnp.exp(s - m)
    return (e * (1.0 / jnp.sum(e, axis=1, keepdims=True))).astype(jnp.bfloat16)


def _compute_body(x_ref, wq_ref, k_ref, v_ref, wo_ref, out_ref, bias_ref):
    h = pl.program_id(0)
    HS = SQ // 2

    @pl.when(h == 0)
    def _():
        rb = lax.broadcasted_iota(jnp.int32, (SQ, SQ), 0) // BLK
        cb = lax.broadcasted_iota(jnp.int32, (SQ, SQ), 1) // BLK
        bias_ref[...] = jnp.where(cb <= rb, 0.0, -1e9).astype(jnp.float32)

    qh = jnp.dot(
        x_ref[...], wq_ref[...], preferred_element_type=jnp.float32
    ).astype(jnp.bfloat16)
    kh = k_ref[...]
    vh = v_ref[...]

    s0 = lax.dot_general(
        qh[:HS], kh[:HS], (((1,), (1,)), ((), ())),
        preferred_element_type=jnp.float32,
    ) + bias_ref[:HS, :HS]
    ctx0 = jnp.dot(
        _softmax_rows(s0), vh[:HS], preferred_element_type=jnp.float32
    ).astype(jnp.bfloat16)

    s1 = lax.dot_general(
        qh[HS:], kh, (((1,), (1,)), ((), ())),
        preferred_element_type=jnp.float32,
    ) + bias_ref[HS:, :]
    ctx1 = jnp.dot(
        _softmax_rows(s1), vh, preferred_element_type=jnp.float32
    ).astype(jnp.bfloat16)

    ctx = jnp.concatenate([ctx0, ctx1], axis=0)
    contrib = jnp.dot(
        ctx, wo_ref[0], preferred_element_type=jnp.float32
    )

    @pl.when(h == 0)
    def _():
        out_ref[...] = contrib

    @pl.when(h != 0)
    def _():
        out_ref[...] += contrib


def _compute_partial(xb, Wqb, Kb, Vb, Wob):
    return pl.pallas_call(
        _compute_body,
        grid=(HEADS_PER,),
        out_shape=jax.ShapeDtypeStruct((SQ, D_MODEL), jnp.float32),
        in_specs=[
            pl.BlockSpec((SQ, D_MODEL), lambda h: (0, 0)),
            pl.BlockSpec((D_MODEL, DH), lambda h: (0, h)),
            pl.BlockSpec((SQ, DH), lambda h: (0, h)),
            pl.BlockSpec((SQ, DH), lambda h: (0, h)),
            pl.BlockSpec((1, DH, D_MODEL), lambda h: (h, 0, 0)),
        ],
        out_specs=pl.BlockSpec((SQ, D_MODEL), lambda h: (0, 0)),
        scratch_shapes=[pltpu.VMEM((SQ, SQ), jnp.float32)],
    )(xb, Wqb, Kb, Vb, Wob)


def kernel(x, Wq, K_ext, V_ext, Wo):
    my = lax.axis_index("i")

    xb = x[0].astype(jnp.bfloat16)
    Wqb = (Wq * SCALE).astype(jnp.bfloat16)

    K = lax.dynamic_slice_in_dim(K_ext[0], my * HEADS_PER, HEADS_PER, axis=1)
    V = lax.dynamic_slice_in_dim(V_ext[0], my * HEADS_PER, HEADS_PER, axis=1)
    Kb = K.astype(jnp.bfloat16).reshape(SQ, HEADS_PER * DH)
    Vb = V.astype(jnp.bfloat16).reshape(SQ, HEADS_PER * DH)
    Wob = Wo.reshape(HEADS_PER, DH, D_MODEL).astype(jnp.bfloat16)

    partial = _compute_partial(xb, Wqb, Kb, Vb, Wob)
    out = _butterfly_allreduce(partial)
    return out[None, :, :]


# device time: 94703 ns/iter; 1.3253x vs baseline; 1.1036x over previous
import jax
import jax.numpy as jnp
from jax import lax
from jax.experimental import pallas as pl
from jax.experimental.pallas import tpu as pltpu

N_DEV = 32
HEADS_PER = 8
DH = 128
SQ = 1024
D_MODEL = 1024
BLK = 64
SCALE = 0.08838834764831843
HS = SQ // 2

N_STEPS = 5
HALF = [256, 128, 64, 32, 16]
OFFS = [0, 256, 384, 448, 480]
COMM_ROWS = 496
ORDER_A = [0, 1, 2, 3, 4]
ORDER_B = [1, 0, 4, 2, 3]


def _lid(cx, cy, cz):
    return 8 * cz + 2 * cy + jnp.bitwise_xor(cx, jnp.bitwise_and(cy, 1))


def _softmax_rows(s):
    m = jnp.max(s, axis=1, keepdims=True)
    e = jnp.exp(s - m)
    return (e * (1.0 / jnp.sum(e, axis=1, keepdims=True))).astype(jnp.bfloat16)


class _Butterfly:

    def __init__(self, base, order, dims, out_ref, comm, agcomm, stage,
                 rs_send, rs_recv, ag_send, ag_recv):
        self.base = base
        self.order = order
        self.dims = dims
        self.out_ref = out_ref
        self.comm = comm
        self.agcomm = agcomm
        self.stage = stage
        self.rs_send = rs_send
        self.rs_recv = rs_recv
        self.ag_send = ag_send
        self.ag_recv = ag_recv
        self.seg = 0
        self.keep = 0
        self.rdma = None

    def _exchange(self, src_start, k, dst, send_sems, recv_sems):
        partner, _ = self.dims[self.order[k]]
        n = HALF[k]
        self.stage[pl.ds(0, n), :] = self.out_ref[
            pl.ds(pl.multiple_of(src_start, 16), n), :].astype(jnp.bfloat16)
        rdma = pltpu.make_async_remote_copy(
            src_ref=self.stage.at[pl.ds(0, n)],
            dst_ref=dst.at[pl.ds(OFFS[k], n)],
            send_sem=send_sems.at[k],
            recv_sem=recv_sems.at[k],
            device_id=(partner,),
            device_id_type=pl.DeviceIdType.MESH,
        )
        rdma.start()
        self.rdma = rdma

    def rs_start(self, k):
        _, b = self.dims[self.order[k]]
        n = HALF[k]
        self._exchange(self.base + self.seg + (1 - b) * n, k,
                       self.comm, self.rs_send, self.rs_recv)
        self.keep = self.seg + b * n

    def rs_finish(self, k):
        n = HALF[k]
        self.rdma.wait()
        start = pl.multiple_of(self.base + self.keep, 16)
        self.out_ref[pl.ds(start, n), :] = (
            self.out_ref[pl.ds(start, n), :]
            + self.comm[pl.ds(OFFS[k], n), :].astype(jnp.float32))
        self.seg = self.keep

    def ag_start(self, k):
        self._exchange(self.base + self.seg, k,
                       self.agcomm, self.ag_send, self.ag_recv)

    def ag_finish(self, k):
        _, b = self.dims[self.order[k]]
        n = HALF[k]
        self.rdma.wait()
        partner_start = pl.multiple_of(
            self.base + self.seg + (1 - 2 * b) * n, 16)
        self.out_ref[pl.ds(partner_start, n), :] = self.agcomm[
            pl.ds(OFFS[k], n), :].astype(jnp.float32)
        self.seg = self.seg - b * n


def _fused_body(x_ref, wq_ref, k_ref, v_ref, wo_ref, out_ref,
                commA, agcommA, stageA, commB, agcommB, stageB,
                bias0_ref, bias1_ref,
                rsA_send, rsA_recv, agA_send, agA_recv,
                rsB_send, rsB_recv, agB_send, agB_recv):
    my = lax.axis_index("i")
    z = my // 8
    p = my % 8
    y = p // 2
    x = (p + y) % 2

    dims = [
        (_lid(1 - x, y, z), x),
        (_lid(x, jnp.bitwise_xor(y, 1), z), jnp.bitwise_and(y, 1)),
        (_lid(x, y, jnp.bitwise_xor(z, 1)), jnp.bitwise_and(z, 1)),
        (_lid(x, jnp.bitwise_xor(y, 2), z), y // 2),
        (_lid(x, y, jnp.bitwise_xor(z, 2)), z // 2),
    ]

    barrier_sem = pltpu.get_barrier_semaphore()
    for partner, _ in dims:
        pl.semaphore_signal(
            barrier_sem, inc=1,
            device_id=(partner,), device_id_type=pl.DeviceIdType.MESH,
        )
    pl.semaphore_wait(barrier_sem, N_STEPS)

    rb = lax.broadcasted_iota(jnp.int32, (HS, HS), 0) // BLK
    cb = lax.broadcasted_iota(jnp.int32, (HS, HS), 1) // BLK
    bias0_ref[...] = jnp.where(cb <= rb, 0.0, -1e9).astype(jnp.float32)
    rb1 = (lax.broadcasted_iota(jnp.int32, (HS, SQ), 0) + HS) // BLK
    cb1 = lax.broadcasted_iota(jnp.int32, (HS, SQ), 1) // BLK
    bias1_ref[...] = jnp.where(cb1 <= rb1, 0.0, -1e9).astype(jnp.float32)

    def compute_half(row0, kv_len, bias_ref):
        xr = x_ref[pl.ds(row0, HS), :]
        for h in range(HEADS_PER):
            c = h * DH
            qh = jnp.dot(
                xr, wq_ref[:, pl.ds(c, DH)],
                preferred_element_type=jnp.float32,
            ).astype(jnp.bfloat16)
            s = lax.dot_general(
                qh, k_ref[pl.ds(0, kv_len), pl.ds(c, DH)],
                (((1,), (1,)), ((), ())),
                preferred_element_type=jnp.float32,
            ) + bias_ref[...]
            ctx = jnp.dot(
                _softmax_rows(s), v_ref[pl.ds(0, kv_len), pl.ds(c, DH)],
                preferred_element_type=jnp.float32,
            ).astype(jnp.bfloat16)
            contrib = jnp.dot(
                ctx, wo_ref[pl.ds(c, DH), :],
                preferred_element_type=jnp.float32,
            )
            if h == 0:
                out_ref[pl.ds(row0, HS), :] = contrib
            else:
                out_ref[pl.ds(row0, HS), :] += contrib

    bflyA = _Butterfly(0, ORDER_A, dims, out_ref, commA, agcommA, stageA,
                       rsA_send, rsA_recv, agA_send, agA_recv)
    bflyB = _Butterfly(HS, ORDER_B, dims, out_ref, commB, agcommB, stageB,
                       rsB_send, rsB_recv, agB_send, agB_recv)

    compute_half(0, HS, bias0_ref)
    bflyA.rs_start(0)
    compute_half(HS, SQ, bias1_ref)
    bflyB.rs_start(0)

    for k in range(N_STEPS):
        bflyA.rs_finish(k)
        if k < N_STEPS - 1:
            bflyA.rs_start(k + 1)
        else:
            bflyA.ag_start(N_STEPS - 1)
        bflyB.rs_finish(k)
        if k < N_STEPS - 1:
            bflyB.rs_start(k + 1)
        else:
            bflyB.ag_start(N_STEPS - 1)

    for k in reversed(range(N_STEPS)):
        bflyA.ag_finish(k)
        if k > 0:
            bflyA.ag_start(k - 1)
        bflyB.ag_finish(k)
        if k > 0:
            bflyB.ag_start(k - 1)


def kernel(x, Wq, K_ext, V_ext, Wo):
    my = lax.axis_index("i")

    xb = x[0].astype(jnp.bfloat16)
    Wqb = (Wq * SCALE).astype(jnp.bfloat16)

    K = lax.dynamic_slice_in_dim(K_ext[0], my * HEADS_PER, HEADS_PER, axis=1)
    V = lax.dynamic_slice_in_dim(V_ext[0], my * HEADS_PER, HEADS_PER, axis=1)
    Kb = K.astype(jnp.bfloat16).reshape(SQ, HEADS_PER * DH)
    Vb = V.astype(jnp.bfloat16).reshape(SQ, HEADS_PER * DH)
    Wob = Wo.astype(jnp.bfloat16)

    bf16_buf = lambda rows: pltpu.VMEM((rows, D_MODEL), jnp.bfloat16)
    sems = pltpu.SemaphoreType.DMA((N_STEPS,))
    vmem = pl.BlockSpec(memory_space=pltpu.VMEM)
    out = pl.pallas_call(
        _fused_body,
        out_shape=jax.ShapeDtypeStruct((SQ, D_MODEL), jnp.float32),
        in_specs=[vmem] * 5,
        out_specs=vmem,
        scratch_shapes=[
            bf16_buf(COMM_ROWS), bf16_buf(COMM_ROWS), bf16_buf(HALF[0]),
            bf16_buf(COMM_ROWS), bf16_buf(COMM_ROWS), bf16_buf(HALF[0]),
            pltpu.VMEM((HS, HS), jnp.float32),
            pltpu.VMEM((HS, SQ), jnp.float32),
            sems, sems, sems, sems,
            sems, sems, sems, sems,
        ],
        compiler_params=pltpu.CompilerParams(collective_id=0),
    )(xb, Wqb, Kb, Vb, Wob)
    return out[None, :, :]


# device time: 76006 ns/iter; 1.6513x vs baseline; 1.2460x over previous
import jax
import jax.numpy as jnp
from jax import lax
from jax.experimental import pallas as pl
from jax.experimental.pallas import tpu as pltpu

N_DEV = 32
HEADS_PER = 8
DH = 128
SQ = 1024
D_MODEL = 1024
BLK = 64
SCALE = 0.08838834764831843
HS = SQ // 2

N_STEPS = 5
HALF = [256, 128, 64, 32, 16]
OFFS = [0, 256, 384, 448, 480]
COMM_ROWS = 496
ORDER_A = [0, 1, 2, 3, 4]
ORDER_B = [1, 0, 4, 2, 3]


def _lid(cx, cy, cz):
    return 8 * cz + 2 * cy + jnp.bitwise_xor(cx, jnp.bitwise_and(cy, 1))


def _softmax_rows(s):
    e = jnp.exp(s)
    return (e * (1.0 / jnp.sum(e, axis=1, keepdims=True))).astype(jnp.bfloat16)


class _Butterfly:

    def __init__(self, base, order, dims, out_ref, comm, agcomm, stage,
                 rs_send, rs_recv, ag_send, ag_recv):
        self.base = base
        self.order = order
        self.dims = dims
        self.out_ref = out_ref
        self.comm = comm
        self.agcomm = agcomm
        self.stage = stage
        self.rs_send = rs_send
        self.rs_recv = rs_recv
        self.ag_send = ag_send
        self.ag_recv = ag_recv
        self.seg = 0
        self.keep = 0
        self.rdma = None

    def _exchange(self, src_start, k, dst, send_sems, recv_sems):
        partner, _ = self.dims[self.order[k]]
        n = HALF[k]
        self.stage[pl.ds(0, n), :] = self.out_ref[
            pl.ds(pl.multiple_of(src_start, 16), n), :].astype(jnp.bfloat16)
        rdma = pltpu.make_async_remote_copy(
            src_ref=self.stage.at[pl.ds(0, n)],
            dst_ref=dst.at[pl.ds(OFFS[k], n)],
            send_sem=send_sems.at[k],
            recv_sem=recv_sems.at[k],
            device_id=(partner,),
            device_id_type=pl.DeviceIdType.MESH,
        )
        rdma.start()
        self.rdma = rdma

    def rs_start(self, k):
        _, b = self.dims[self.order[k]]
        n = HALF[k]
        self._exchange(self.base + self.seg + (1 - b) * n, k,
                       self.comm, self.rs_send, self.rs_recv)
        self.keep = self.seg + b * n

    def rs_finish(self, k):
        n = HALF[k]
        self.rdma.wait()
        start = pl.multiple_of(self.base + self.keep, 16)
        self.out_ref[pl.ds(start, n), :] = (
            self.out_ref[pl.ds(start, n), :]
            + self.comm[pl.ds(OFFS[k], n), :].astype(jnp.float32))
        self.seg = self.keep

    def ag_start(self, k):
        self._exchange(self.base + self.seg, k,
                       self.agcomm, self.ag_send, self.ag_recv)

    def ag_finish(self, k):
        _, b = self.dims[self.order[k]]
        n = HALF[k]
        self.rdma.wait()
        partner_start = pl.multiple_of(
            self.base + self.seg + (1 - 2 * b) * n, 16)
        self.out_ref[pl.ds(partner_start, n), :] = self.agcomm[
            pl.ds(OFFS[k], n), :].astype(jnp.float32)
        self.seg = self.seg - b * n


def _fused_body(x_ref, wq_ref, k_ref, v_ref, wo_ref, out_ref,
                commA, agcommA, stageA, commB, agcommB, stageB,
                bias0_ref, bias1_ref, ctx_ref,
                rsA_send, rsA_recv, agA_send, agA_recv,
                rsB_send, rsB_recv, agB_send, agB_recv):
    my = lax.axis_index("i")
    z = my // 8
    p = my % 8
    y = p // 2
    x = (p + y) % 2

    dims = [
        (_lid(1 - x, y, z), x),
        (_lid(x, jnp.bitwise_xor(y, 1), z), jnp.bitwise_and(y, 1)),
        (_lid(x, y, jnp.bitwise_xor(z, 1)), jnp.bitwise_and(z, 1)),
        (_lid(x, jnp.bitwise_xor(y, 2), z), y // 2),
        (_lid(x, y, jnp.bitwise_xor(z, 2)), z // 2),
    ]

    barrier_sem = pltpu.get_barrier_semaphore()
    for partner, _ in dims:
        pl.semaphore_signal(
            barrier_sem, inc=1,
            device_id=(partner,), device_id_type=pl.DeviceIdType.MESH,
        )
    pl.semaphore_wait(barrier_sem, N_STEPS)

    rb = lax.broadcasted_iota(jnp.int32, (HS, HS), 0) // BLK
    cb = lax.broadcasted_iota(jnp.int32, (HS, HS), 1) // BLK
    bias0_ref[...] = jnp.where(cb <= rb, 0.0, -1e9).astype(jnp.float32)
    rb1 = (lax.broadcasted_iota(jnp.int32, (HS, SQ), 0) + HS) // BLK
    cb1 = lax.broadcasted_iota(jnp.int32, (HS, SQ), 1) // BLK
    bias1_ref[...] = jnp.where(cb1 <= rb1, 0.0, -1e9).astype(jnp.float32)

    def compute_half(row0, kv_len, bias_ref):
        q_full = jnp.dot(
            x_ref[pl.ds(row0, HS), :], wq_ref[...],
            preferred_element_type=jnp.float32,
        ).astype(jnp.bfloat16)
        for h in range(HEADS_PER):
            c = h * DH
            s = lax.dot_general(
                q_full[:, c:c + DH], k_ref[pl.ds(0, kv_len), pl.ds(c, DH)],
                (((1,), (1,)), ((), ())),
                preferred_element_type=jnp.float32,
            ) + bias_ref[...]
            ctx_ref[:, pl.ds(c, DH)] = jnp.dot(
                _softmax_rows(s), v_ref[pl.ds(0, kv_len), pl.ds(c, DH)],
                preferred_element_type=jnp.float32,
            ).astype(jnp.bfloat16)
        out_ref[pl.ds(row0, HS), :] = jnp.dot(
            ctx_ref[...], wo_ref[...], preferred_element_type=jnp.float32)

    bflyA = _Butterfly(0, ORDER_A, dims, out_ref, commA, agcommA, stageA,
                       rsA_send, rsA_recv, agA_send, agA_recv)
    bflyB = _Butterfly(HS, ORDER_B, dims, out_ref, commB, agcommB, stageB,
                       rsB_send, rsB_recv, agB_send, agB_recv)

    compute_half(0, HS, bias0_ref)
    bflyA.rs_start(0)
    compute_half(HS, SQ, bias1_ref)
    bflyB.rs_start(0)

    for k in range(N_STEPS):
        bflyA.rs_finish(k)
        if k < N_STEPS - 1:
            bflyA.rs_start(k + 1)
        else:
            bflyA.ag_start(N_STEPS - 1)
        bflyB.rs_finish(k)
        if k < N_STEPS - 1:
            bflyB.rs_start(k + 1)
        else:
            bflyB.ag_start(N_STEPS - 1)

    for k in reversed(range(N_STEPS)):
        bflyA.ag_finish(k)
        if k > 0:
            bflyA.ag_start(k - 1)
        bflyB.ag_finish(k)
        if k > 0:
            bflyB.ag_start(k - 1)


def kernel(x, Wq, K_ext, V_ext, Wo):
    my = lax.axis_index("i")

    xb = x[0].astype(jnp.bfloat16)
    Wqb = (Wq * SCALE).astype(jnp.bfloat16)

    K = lax.dynamic_slice_in_dim(K_ext[0], my * HEADS_PER, HEADS_PER, axis=1)
    V = lax.dynamic_slice_in_dim(V_ext[0], my * HEADS_PER, HEADS_PER, axis=1)
    Kb = K.astype(jnp.bfloat16).reshape(SQ, HEADS_PER * DH)
    Vb = V.astype(jnp.bfloat16).reshape(SQ, HEADS_PER * DH)
    Wob = Wo.astype(jnp.bfloat16)

    bf16_buf = lambda rows: pltpu.VMEM((rows, D_MODEL), jnp.bfloat16)
    sems = pltpu.SemaphoreType.DMA((N_STEPS,))
    vmem = pl.BlockSpec(memory_space=pltpu.VMEM)
    out = pl.pallas_call(
        _fused_body,
        out_shape=jax.ShapeDtypeStruct((SQ, D_MODEL), jnp.float32),
        in_specs=[vmem] * 5,
        out_specs=vmem,
        scratch_shapes=[
            bf16_buf(COMM_ROWS), bf16_buf(COMM_ROWS), bf16_buf(HALF[0]),
            bf16_buf(COMM_ROWS), bf16_buf(COMM_ROWS), bf16_buf(HALF[0]),
            pltpu.VMEM((HS, HS), jnp.float32),
            pltpu.VMEM((HS, SQ), jnp.float32),
            pltpu.VMEM((HS, D_MODEL), jnp.bfloat16),
            sems, sems, sems, sems,
            sems, sems, sems, sems,
        ],
        compiler_params=pltpu.CompilerParams(collective_id=0),
    )(xb, Wqb, Kb, Vb, Wob)
    return out[None, :, :]


# device time: 71788 ns/iter; 1.7484x vs baseline; 1.0588x over previous
import jax
import jax.numpy as jnp
from jax import lax
from jax.experimental import pallas as pl
from jax.experimental.pallas import tpu as pltpu

N_DEV = 32
HEADS_PER = 8
DH = 128
SQ = 1024
D_MODEL = 1024
BLK = 64
SCALE = 0.08838834764831843
HS = SQ // 2

N_STEPS = 5
HALF = [256, 128, 64, 32, 16]
OFFS = [0, 256, 384, 448, 480]
COMM_ROWS = 496
ORDER_A = [0, 1, 2, 3, 4]
ORDER_B = [1, 0, 4, 2, 3]


def _lid(cx, cy, cz):
    return 8 * cz + 2 * cy + jnp.bitwise_xor(cx, jnp.bitwise_and(cy, 1))


def _softmax_rows(s):
    e = jnp.exp(s)
    return (e * (1.0 / jnp.sum(e, axis=1, keepdims=True))).astype(jnp.bfloat16)


class _Butterfly:

    def __init__(self, base, order, dims, out_ref, comm, agcomm, stage,
                 rs_send, rs_recv, ag_send, ag_recv):
        self.base = base
        self.order = order
        self.dims = dims
        self.out_ref = out_ref
        self.comm = comm
        self.agcomm = agcomm
        self.stage = stage
        self.rs_send = rs_send
        self.rs_recv = rs_recv
        self.ag_send = ag_send
        self.ag_recv = ag_recv
        self.seg = 0
        self.keep = 0
        self.rdma = None

    def _exchange(self, src_start, k, dst, send_sems, recv_sems):
        partner, _ = self.dims[self.order[k]]
        n = HALF[k]
        self.stage[pl.ds(0, n), :] = self.out_ref[
            pl.ds(pl.multiple_of(src_start, 16), n), :].astype(jnp.bfloat16)
        rdma = pltpu.make_async_remote_copy(
            src_ref=self.stage.at[pl.ds(0, n)],
            dst_ref=dst.at[pl.ds(OFFS[k], n)],
            send_sem=send_sems.at[k],
            recv_sem=recv_sems.at[k],
            device_id=(partner,),
            device_id_type=pl.DeviceIdType.MESH,
        )
        rdma.start()
        self.rdma = rdma

    def rs_start(self, k):
        _, b = self.dims[self.order[k]]
        n = HALF[k]
        self._exchange(self.base + self.seg + (1 - b) * n, k,
                       self.comm, self.rs_send, self.rs_recv)
        self.keep = self.seg + b * n

    def rs_finish(self, k):
        n = HALF[k]
        self.rdma.wait()
        start = pl.multiple_of(self.base + self.keep, 16)
        self.out_ref[pl.ds(start, n), :] = (
            self.out_ref[pl.ds(start, n), :]
            + self.comm[pl.ds(OFFS[k], n), :].astype(jnp.float32))
        self.seg = self.keep

    def ag_start(self, k):
        self._exchange(self.base + self.seg, k,
                       self.agcomm, self.ag_send, self.ag_recv)

    def ag_finish(self, k):
        _, b = self.dims[self.order[k]]
        n = HALF[k]
        self.rdma.wait()
        partner_start = pl.multiple_of(
            self.base + self.seg + (1 - 2 * b) * n, 16)
        self.out_ref[pl.ds(partner_start, n), :] = self.agcomm[
            pl.ds(OFFS[k], n), :].astype(jnp.float32)
        self.seg = self.seg - b * n


def _fused_body(x_ref, wq_ref, k_hbm, v_hbm, wo_ref, out_ref,
                commA, agcommA, stageA, commB, agcommB, stageB,
                bias0_ref, bias1_ref, ctx_ref, k_ref, v_ref, kv_sems,
                rsA_send, rsA_recv, agA_send, agA_recv,
                rsB_send, rsB_recv, agB_send, agB_recv):
    my = lax.axis_index("i")
    z = my // 8
    p = my % 8
    y = p // 2
    x = (p + y) % 2

    dims = [
        (_lid(1 - x, y, z), x),
        (_lid(x, jnp.bitwise_xor(y, 1), z), jnp.bitwise_and(y, 1)),
        (_lid(x, y, jnp.bitwise_xor(z, 1)), jnp.bitwise_and(z, 1)),
        (_lid(x, jnp.bitwise_xor(y, 2), z), y // 2),
        (_lid(x, y, jnp.bitwise_xor(z, 2)), z // 2),
    ]

    kv_dmas = []
    for h in range(HEADS_PER):
        for slot, (hbm, dst) in enumerate(((k_hbm, k_ref), (v_hbm, v_ref))):
            dma = pltpu.make_async_copy(
                hbm.at[:, my * HEADS_PER + h, :],
                dst.at[:, pl.ds(h * DH, DH)],
                kv_sems.at[2 * h + slot],
            )
            dma.start()
            kv_dmas.append(dma)

    barrier_sem = pltpu.get_barrier_semaphore()
    for partner, _ in dims:
        pl.semaphore_signal(
            barrier_sem, inc=1,
            device_id=(partner,), device_id_type=pl.DeviceIdType.MESH,
        )
    pl.semaphore_wait(barrier_sem, N_STEPS)

    rb = lax.broadcasted_iota(jnp.int32, (HS, HS), 0) // BLK
    cb = lax.broadcasted_iota(jnp.int32, (HS, HS), 1) // BLK
    bias0_ref[...] = jnp.where(cb <= rb, 0.0, -1e9).astype(jnp.float32)
    rb1 = (lax.broadcasted_iota(jnp.int32, (HS, SQ), 0) + HS) // BLK
    cb1 = lax.broadcasted_iota(jnp.int32, (HS, SQ), 1) // BLK
    bias1_ref[...] = jnp.where(cb1 <= rb1, 0.0, -1e9).astype(jnp.float32)

    def compute_half(row0, kv_len, bias_ref):
        q_full = jnp.dot(
            x_ref[pl.ds(row0, HS), :], wq_ref[...],
            preferred_element_type=jnp.float32,
        ).astype(jnp.bfloat16)
        for h in range(HEADS_PER):
            c = h * DH
            kh = k_ref[pl.ds(0, kv_len), pl.ds(c, DH)].astype(jnp.bfloat16)
            vh = v_ref[pl.ds(0, kv_len), pl.ds(c, DH)].astype(jnp.bfloat16)
            s = lax.dot_general(
                q_full[:, c:c + DH], kh,
                (((1,), (1,)), ((), ())),
                preferred_element_type=jnp.float32,
            ) + bias_ref[...]
            ctx_ref[:, pl.ds(c, DH)] = jnp.dot(
                _softmax_rows(s), vh,
                preferred_element_type=jnp.float32,
            ).astype(jnp.bfloat16)
        out_ref[pl.ds(row0, HS), :] = jnp.dot(
            ctx_ref[...], wo_ref[...], preferred_element_type=jnp.float32)

    bflyA = _Butterfly(0, ORDER_A, dims, out_ref, commA, agcommA, stageA,
                       rsA_send, rsA_recv, agA_send, agA_recv)
    bflyB = _Butterfly(HS, ORDER_B, dims, out_ref, commB, agcommB, stageB,
                       rsB_send, rsB_recv, agB_send, agB_recv)

    for dma in kv_dmas:
        dma.wait()

    compute_half(0, HS, bias0_ref)
    bflyA.rs_start(0)
    compute_half(HS, SQ, bias1_ref)
    bflyB.rs_start(0)

    for k in range(N_STEPS):
        bflyA.rs_finish(k)
        if k < N_STEPS - 1:
            bflyA.rs_start(k + 1)
        else:
            bflyA.ag_start(N_STEPS - 1)
        bflyB.rs_finish(k)
        if k < N_STEPS - 1:
            bflyB.rs_start(k + 1)
        else:
            bflyB.ag_start(N_STEPS - 1)

    for k in reversed(range(N_STEPS)):
        bflyA.ag_finish(k)
        if k > 0:
            bflyA.ag_start(k - 1)
        bflyB.ag_finish(k)
        if k > 0:
            bflyB.ag_start(k - 1)


def kernel(x, Wq, K_ext, V_ext, Wo):
    my = lax.axis_index("i")

    xb = x[0].astype(jnp.bfloat16)
    Wqb = (Wq * SCALE).astype(jnp.bfloat16)

    Wob = Wo.astype(jnp.bfloat16)

    bf16_buf = lambda rows: pltpu.VMEM((rows, D_MODEL), jnp.bfloat16)
    sems = pltpu.SemaphoreType.DMA((N_STEPS,))
    vmem = pl.BlockSpec(memory_space=pltpu.VMEM)
    anyspace = pl.BlockSpec(memory_space=pltpu.MemorySpace.HBM)
    out = pl.pallas_call(
        _fused_body,
        out_shape=jax.ShapeDtypeStruct((SQ, D_MODEL), jnp.float32),
        in_specs=[vmem, vmem, anyspace, anyspace, vmem],
        out_specs=vmem,
        scratch_shapes=[
            bf16_buf(COMM_ROWS), bf16_buf(COMM_ROWS), bf16_buf(HALF[0]),
            bf16_buf(COMM_ROWS), bf16_buf(COMM_ROWS), bf16_buf(HALF[0]),
            pltpu.VMEM((HS, HS), jnp.float32),
            pltpu.VMEM((HS, SQ), jnp.float32),
            pltpu.VMEM((HS, D_MODEL), jnp.bfloat16),
            pltpu.VMEM((SQ, HEADS_PER * DH), jnp.float32),
            pltpu.VMEM((SQ, HEADS_PER * DH), jnp.float32),
            pltpu.SemaphoreType.DMA((2 * HEADS_PER,)),
            sems, sems, sems, sems,
            sems, sems, sems, sems,
        ],
        compiler_params=pltpu.CompilerParams(collective_id=0),
    )(xb, Wqb, K_ext[0], V_ext[0], Wob)
    return out[None, :, :]
